# Initial kernel scaffold; baseline (speedup 1.0000x reference)
import jax
import jax.numpy as jnp
from jax.experimental import pallas as pl


def kernel(x, edge_index, edge_attr, batch, weight):
    raise NotImplementedError("write your pallas kernel here")


# speedup vs baseline: 17.0204x; 17.0204x over previous
import functools
import math

import jax
import jax.numpy as jnp
from jax import lax
from jax.experimental import pallas as pl
from jax.experimental.pallas import tpu as pltpu
from jax.experimental.pallas import tpu_sc as plsc

N = 10000
E = 320000
D = 128
DE = 16
K = int(math.ceil(0.5 * N))

_NC = 2
_NS = 16
_NW = _NC * _NS
_EC = E // _NW
_KP = ((K + 15) // 16) * 16
_BP = ((K + 255) // 256) * 256
_BPW = _BP // _NW

_mesh = plsc.VectorSubcoreMesh(
    core_axis_name="c", subcore_axis_name="s", num_cores=_NC, num_subcores=_NS)


def _wid():
    return lax.axis_index("s") * _NC + lax.axis_index("c")


@functools.partial(
    pl.kernel,
    out_type=(
        jax.ShapeDtypeStruct((E,), jnp.int32),
        jax.ShapeDtypeStruct((E,), jnp.int32),
    ),
    mesh=_mesh,
    scratch_types=[
        pltpu.VMEM((N + 16,), jnp.int32),
        pltpu.VMEM((_KP,), jnp.int32),
        pltpu.VMEM((_EC,), jnp.int32),
        pltpu.VMEM((_EC,), jnp.int32),
        pltpu.VMEM((_EC,), jnp.int32),
        pltpu.VMEM((_EC,), jnp.int32),
    ],
    compiler_params=pltpu.CompilerParams(needs_layout_passes=False),
)
def _edge_remap(perm_hbm, src_hbm, dst_hbm, ns_hbm, nd_hbm,
                map_v, perm_v, src_v, dst_v, ns_v, nd_v):
    base = _wid() * _EC
    pltpu.sync_copy(perm_hbm, perm_v)
    pltpu.sync_copy(src_hbm.at[pl.ds(base, _EC)], src_v)
    pltpu.sync_copy(dst_hbm.at[pl.ds(base, _EC)], dst_v)

    def init_body(i, _):
        map_v[pl.ds(i * 16, 16)] = jnp.full((16,), -1, jnp.int32)
        return 0
    lax.fori_loop(0, (N + 16) // 16, init_body, 0)

    def scat_body(i, _):
        pv = perm_v[pl.ds(i * 16, 16)]
        pos = lax.iota(jnp.int32, 16) + i * 16
        plsc.store_scatter(map_v, [pv], pos)
        return 0
    lax.fori_loop(0, _KP // 16, scat_body, 0)

    def edge_body(i, _):
        s = src_v[pl.ds(i * 16, 16)]
        d = dst_v[pl.ds(i * 16, 16)]
        ms = plsc.load_gather(map_v, [s])
        md = plsc.load_gather(map_v, [d])
        keep = (ms >= 0) & (md >= 0)
        neg1 = jnp.full((16,), -1, jnp.int32)
        ns_v[pl.ds(i * 16, 16)] = jnp.where(keep, ms, neg1)
        nd_v[pl.ds(i * 16, 16)] = jnp.where(keep, md, neg1)
        return 0
    lax.fori_loop(0, _EC // 16, edge_body, 0)

    pltpu.sync_copy(ns_v, ns_hbm.at[pl.ds(base, _EC)])
    pltpu.sync_copy(nd_v, nd_hbm.at[pl.ds(base, _EC)])


@functools.partial(
    pl.kernel,
    out_type=jax.ShapeDtypeStruct((_BP, D), jnp.float32),
    mesh=_mesh,
    scratch_types=[
        pltpu.VMEM((_BPW,), jnp.int32),
        pltpu.VMEM((_BPW, D), jnp.float32),
        pltpu.VMEM((_BPW,), jnp.float32),
        pltpu.SemaphoreType.DMA,
    ],
    compiler_params=pltpu.CompilerParams(needs_layout_passes=False),
)
def _row_gather(x_hbm, idx_hbm, ts_hbm, out_hbm, idx_v, rows_v, ts_v, sem):
    base = _wid() * _BPW
    pltpu.sync_copy(idx_hbm.at[pl.ds(base, _BPW)], idx_v)
    pltpu.sync_copy(ts_hbm.at[pl.ds(base, _BPW)], ts_v)
    pltpu.async_copy(x_hbm.at[idx_v], rows_v, sem).wait()

    def row_body(r, _):
        tb = plsc.load_gather(ts_v, [jnp.full((16,), 0, jnp.int32) + r])

        def col_body(j, _):
            rows_v[r, pl.ds(j * 16, 16)] = rows_v[r, pl.ds(j * 16, 16)] * tb
            return 0
        lax.fori_loop(0, D // 16, col_body, 0)
        return 0
    lax.fori_loop(0, _BPW, row_body, 0)

    pltpu.sync_copy(rows_v, out_hbm.at[pl.ds(base, _BPW)])


_BE = 6400


def _mask_body(ea_ref, ns_ref, out_ref):
    keep = (ns_ref[...] >= 0).astype(jnp.float32)
    out_ref[...] = ea_ref[...] * keep


def _mask_edge_attr(edge_attr, new_src):
    return pl.pallas_call(
        _mask_body,
        out_shape=jax.ShapeDtypeStruct((E, DE), jnp.float32),
        grid=(E // _BE,),
        in_specs=[
            pl.BlockSpec((_BE, DE), lambda i: (i, 0)),
            pl.BlockSpec((_BE, 1), lambda i: (i, 0)),
        ],
        out_specs=pl.BlockSpec((_BE, DE), lambda i: (i, 0)),
    )(edge_attr, new_src)


def kernel(x, edge_index, edge_attr, batch, weight):
    score = x @ weight
    t = jnp.tanh(score / jnp.linalg.norm(weight))
    topk_scores, perm = jax.lax.top_k(t, K)

    perm_pad = jnp.pad(perm, (0, _KP - K), constant_values=N)
    new_src, new_dst = _edge_remap(perm_pad, edge_index[0], edge_index[1])
    new_edge_index = jnp.stack([new_src, new_dst])

    idx_pad = jnp.pad(perm, (0, _BP - K))
    ts_pad = jnp.pad(topk_scores, (0, _BP - K))
    x_out = _row_gather(x, idx_pad, ts_pad)[:K]

    new_edge_attr = _mask_edge_attr(edge_attr, new_src.reshape(E, 1))
    batch_out = jnp.zeros((K,), dtype=batch.dtype)
    return x_out, new_edge_index, new_edge_attr, batch_out
